# baseline (speedup 1.0000x reference)
import functools

import jax
import jax.numpy as jnp
import numpy as np
from jax import lax
from jax.experimental import pallas as pl
from jax.experimental.pallas import tpu as pltpu
from jax.experimental.pallas import tpu_sc as plsc

N_NODES = 10000
IN_DIM = 128
NUM_HEADS = 8
HEAD_DIM = 16
HD = NUM_HEADS * HEAD_DIM
N_EDGES = 320000
HPC = NUM_HEADS // 2
CW = HPC * HEAD_DIM
KVW = 2 * CW
ROW = 80
CHUNK = 80
N_CORES = 2
N_SUBCORES = 16
EDGES_PER_TILE = N_EDGES // N_SUBCORES
N_CHUNKS = EDGES_PER_TILE // CHUNK
ACC_ROWS = 10240
ROWS_PER_SUB = ACC_ROWS // N_SUBCORES
NBUF = 3


def _qkv_body(h_ref, wq_ref, wk_ref, wv_ref, q_ref, kv_ref):
    hb = h_ref[...]
    q_ref[...] = jnp.dot(hb, wq_ref[...][0], preferred_element_type=jnp.float32)
    kb = jnp.dot(hb, wk_ref[...][0], preferred_element_type=jnp.float32)
    vb = jnp.dot(hb, wv_ref[...][0], preferred_element_type=jnp.float32)
    kv_ref[...] = jnp.concatenate([kb, vb], axis=1)


def _qkv(h, WQ, WK, WV):
    blk = 1000
    nb = N_NODES // blk
    w_spec = pl.BlockSpec((1, IN_DIM, CW), lambda i, c: (c, 0, 0))
    return pl.pallas_call(
        _qkv_body,
        grid=(nb, N_CORES),
        in_specs=[pl.BlockSpec((blk, IN_DIM), lambda i, c: (i, 0)),
                  w_spec, w_spec, w_spec],
        out_specs=[
            pl.BlockSpec((blk, CW), lambda i, c: (c * nb + i, 0)),
            pl.BlockSpec((blk, KVW), lambda i, c: (c * nb + i, 0)),
        ],
        out_shape=[
            jax.ShapeDtypeStruct((N_CORES * N_NODES, CW), jnp.float32),
            jax.ShapeDtypeStruct((N_CORES * N_NODES, KVW), jnp.float32),
        ],
    )(h, WQ, WK, WV)


def _edge_body(q_hbm, kv_hbm, ei_hbm, out_hbm,
               ebuf, didx, gsidx, gdidx, kvbuf, qbuf, wvbuf,
               zerobuf, accum, sem_g, sem_s):
    cid = lax.axis_index("c")
    sid = lax.axis_index("s")
    node_off = cid * N_NODES

    zeros16 = jnp.zeros((16,), jnp.float32)

    def _zrow(r, _):
        for c in range(ROW // 16):
            zerobuf[r, pl.ds(c * 16, 16)] = zeros16
        return 0
    lax.fori_loop(0, CHUNK, _zrow, 0)

    base = sid * ROWS_PER_SUB
    for b in range(ROWS_PER_SUB // CHUNK):
        pltpu.sync_copy(zerobuf, accum.at[pl.ds(base + b * CHUNK, CHUNK)])
    plsc.subcore_barrier()

    edge_base = sid * EDGES_PER_TILE
    iota16 = lax.iota(jnp.int32, 16)

    def _start_gathers(i, b):
        off = edge_base + i * CHUNK
        pltpu.sync_copy(ei_hbm.at[:, pl.ds(off, CHUNK)], ebuf[b])
        for j in range(CHUNK // 16):
            sl = pl.ds(j * 16, 16)
            s_v = ebuf[b][0, sl]
            d_v = ebuf[b][1, sl]
            gsidx[b][sl] = s_v + node_off
            gdidx[b][sl] = d_v + node_off
            didx[b][sl] = d_v
        pltpu.async_copy(kv_hbm.at[gsidx[b]], kvbuf[b], sem_g[b])
        pltpu.async_copy(q_hbm.at[gdidx[b]], qbuf[b], sem_g[b])

    def _wait_gathers(b):
        pltpu.make_async_copy(kv_hbm.at[gsidx[b]], kvbuf[b], sem_g[b]).wait()
        pltpu.make_async_copy(q_hbm.at[gdidx[b]], qbuf[b], sem_g[b]).wait()

    lanesel = [(iota16 % 4) == hh for hh in range(1, HPC)]

    def _compute(b):
        def _edge(e2, _):
            for u in range(4):
                e = 4 * e2 + u
                raws = []
                for hh in range(HPC):
                    cb = hh * HEAD_DIM
                    kvv = kvbuf[b][e, pl.ds(cb, 16)]
                    qvv = qbuf[b][e, pl.ds(cb, 16)]
                    raws.append(jnp.sum(kvv * qvv))
                comb = jnp.full((16,), raws[0], jnp.float32)
                for hh in range(1, HPC):
                    comb = jnp.where(lanesel[hh - 1],
                                     jnp.full((16,), raws[hh], jnp.float32), comb)
                s_all = jnp.exp(jnp.clip(comb, -5.0, 5.0))
                wvbuf[b][e, pl.ds(CW, 16)] = s_all
                for k in range(4):
                    vv = kvbuf[b][e, pl.ds(CW + 16 * k, 16)]
                    wvbuf[b][e, pl.ds(16 * k, 16)] = vv * s_all
            return 0

        lax.fori_loop(0, CHUNK // 4, _edge, 0)

    def _wait_scatter(b):
        pltpu.make_async_copy(wvbuf[b], accum.at[didx[b]], sem_s[b]).wait()

    def _issue_scatter(b):
        pltpu.async_copy(wvbuf[b], accum.at[didx[b]], sem_s[b], add=True)

    _start_gathers(0, 0)
    _start_gathers(1, 1)

    NSUP = N_CHUNKS // NBUF

    def _super(t, _):
        i0 = t * NBUF
        for b in range(NBUF):
            _wait_gathers(b)
            _compute(b)
            _issue_scatter(b)
            bj = (b + 2) % NBUF
            if b == 0:
                @pl.when(t > 0)
                def _():
                    _wait_scatter(bj)
                _start_gathers(i0 + 2, bj)
            elif b == NBUF - 1:
                @pl.when(t < NSUP - 1)
                def _():
                    _wait_scatter(bj)
                    _start_gathers(i0 + b + 2, bj)
            else:
                _wait_scatter(bj)
                _start_gathers(i0 + b + 2, bj)
        return 0

    lax.fori_loop(0, NSUP, _super, 0)
    _wait_gathers(0)
    _compute(0)
    _issue_scatter(0)
    for b in range(NBUF):
        _wait_scatter(b)
    plsc.subcore_barrier()

    for b in range(5):
        rs = pl.ds(base + b * 128, 128)
        pltpu.sync_copy(accum.at[rs], out_hbm.at[cid, rs])


def _edges(q2, kv2, ei32):
    mesh = plsc.VectorSubcoreMesh(core_axis_name="c", subcore_axis_name="s")
    idx_t = pltpu.VMEM((CHUNK,), jnp.int32)
    f = functools.partial(
        pl.kernel,
        out_type=jax.ShapeDtypeStruct((N_CORES, ACC_ROWS, ROW), jnp.float32),
        mesh=mesh,
        compiler_params=pltpu.CompilerParams(
            needs_layout_passes=False, use_tc_tiling_on_sc=False),
        scratch_types=[
            [pltpu.VMEM((2, CHUNK), jnp.int32)] * NBUF,
            [idx_t] * NBUF,
            [idx_t] * NBUF,
            [idx_t] * NBUF,
            [pltpu.VMEM((CHUNK, KVW), jnp.float32)] * NBUF,
            [pltpu.VMEM((CHUNK, CW), jnp.float32)] * NBUF,
            [pltpu.VMEM((CHUNK, ROW), jnp.float32)] * NBUF,
            pltpu.VMEM((CHUNK, ROW), jnp.float32),
            pltpu.VMEM_SHARED((ACC_ROWS, ROW), jnp.float32),
            [pltpu.SemaphoreType.DMA] * NBUF,
            [pltpu.SemaphoreType.DMA] * NBUF,
        ],
    )(_edge_body)
    return f(q2, kv2, ei32)


def _perm_core():
    return np.array([16 * (cp % 4) + 4 * (cp // 16) + (cp % 16) // 4
                     for cp in range(CW)])


def _comb_body(p0_ref, p1_ref, b_ref, pm_ref, o_ref):
    bm = b_ref[...]
    pm = pm_ref[...]
    s0 = p0_ref[...][0]
    s1 = p1_ref[...][0]
    z0 = jnp.dot(s0[:, CW:CW + 16], bm, preferred_element_type=jnp.float32)
    z1 = jnp.dot(s1[:, CW:CW + 16], bm, preferred_element_type=jnp.float32)
    o0 = jnp.dot(s0[:, :CW] / z0, pm, preferred_element_type=jnp.float32)
    o1 = jnp.dot(s1[:, :CW] / z1, pm, preferred_element_type=jnp.float32)
    o_ref[...] = jnp.concatenate([o0, o1], axis=1)


def _combine(partials):
    blk = 1000
    perm = _perm_core()
    bm_np = np.zeros((16, CW), np.float32)
    for cp in range(CW):
        bm_np[cp % 4, cp] = 1.0
    pm_np = np.zeros((CW, CW), np.float32)
    for cp in range(CW):
        pm_np[cp, perm[cp]] = 1.0
    return pl.pallas_call(
        _comb_body,
        grid=(N_NODES // blk,),
        in_specs=[
            pl.BlockSpec((1, blk, ROW), lambda i: (0, i, 0)),
            pl.BlockSpec((1, blk, ROW), lambda i: (1, i, 0)),
            pl.BlockSpec((16, CW), lambda i: (0, 0)),
            pl.BlockSpec((CW, CW), lambda i: (0, 0)),
        ],
        out_specs=pl.BlockSpec((blk, HD), lambda i: (i, 0)),
        out_shape=jax.ShapeDtypeStruct((N_NODES, HD), jnp.float32),
    )(partials, partials, jnp.asarray(bm_np), jnp.asarray(pm_np))


def kernel(h, edge_index, WQ, WK, WV):
    ei32 = edge_index.astype(jnp.int32)
    wq = (WQ * 0.25).reshape(IN_DIM, N_CORES, CW).transpose(1, 0, 2)
    wk = WK.reshape(IN_DIM, N_CORES, CW).transpose(1, 0, 2)
    pc = _perm_core()
    wv = WV[:, np.concatenate([pc, CW + pc])]
    wv = wv.reshape(IN_DIM, N_CORES, CW).transpose(1, 0, 2)
    q2, kv2 = _qkv(h, wq, wk, wv)
    partials = _edges(q2, kv2, ei32)
    out = _combine(partials)
    return out.reshape(N_NODES, NUM_HEADS, HEAD_DIM)


# speedup vs baseline: 1.0022x; 1.0022x over previous
import functools

import jax
import jax.numpy as jnp
import numpy as np
from jax import lax
from jax.experimental import pallas as pl
from jax.experimental.pallas import tpu as pltpu
from jax.experimental.pallas import tpu_sc as plsc

N_NODES = 10000
IN_DIM = 128
NUM_HEADS = 8
HEAD_DIM = 16
HD = NUM_HEADS * HEAD_DIM
N_EDGES = 320000
HPC = NUM_HEADS // 2
CW = HPC * HEAD_DIM
KVW = 2 * CW
ROW = 80
CHUNK = 80
N_CORES = 2
N_SUBCORES = 16
EDGES_PER_TILE = N_EDGES // N_SUBCORES
N_CHUNKS = EDGES_PER_TILE // CHUNK
ACC_ROWS = 10240
ROWS_PER_SUB = ACC_ROWS // N_SUBCORES
NBUF = 3


def _qkv_body(h_ref, wq_ref, wk_ref, wv_ref, q_ref, kv_ref):
    hb = h_ref[...]
    q_ref[...] = jnp.dot(hb, wq_ref[...][0], preferred_element_type=jnp.float32)
    kb = jnp.dot(hb, wk_ref[...][0], preferred_element_type=jnp.float32)
    vb = jnp.dot(hb, wv_ref[...][0], preferred_element_type=jnp.float32)
    kv_ref[...] = jnp.concatenate([kb, vb], axis=1)


def _qkv(h, WQ, WK, WV):
    blk = 1000
    nb = N_NODES // blk
    w_spec = pl.BlockSpec((1, IN_DIM, CW), lambda i, c: (c, 0, 0))
    return pl.pallas_call(
        _qkv_body,
        grid=(nb, N_CORES),
        in_specs=[pl.BlockSpec((blk, IN_DIM), lambda i, c: (i, 0)),
                  w_spec, w_spec, w_spec],
        out_specs=[
            pl.BlockSpec((blk, CW), lambda i, c: (c * nb + i, 0)),
            pl.BlockSpec((blk, KVW), lambda i, c: (c * nb + i, 0)),
        ],
        out_shape=[
            jax.ShapeDtypeStruct((N_CORES * N_NODES, CW), jnp.float32),
            jax.ShapeDtypeStruct((N_CORES * N_NODES, KVW), jnp.float32),
        ],
    )(h, WQ, WK, WV)


def _edge_body(q_hbm, kv_hbm, ei_hbm, out_hbm,
               ebuf, didx, gsidx, gdidx, kvbuf, qbuf, wvbuf,
               zerobuf, accum, sem_g, sem_s):
    cid = lax.axis_index("c")
    sid = lax.axis_index("s")
    node_off = cid * N_NODES

    zeros16 = jnp.zeros((16,), jnp.float32)

    def _zrow(r, _):
        for c in range(ROW // 16):
            zerobuf[r, pl.ds(c * 16, 16)] = zeros16
        return 0
    lax.fori_loop(0, CHUNK, _zrow, 0)

    base = sid * ROWS_PER_SUB
    for b in range(ROWS_PER_SUB // CHUNK):
        pltpu.sync_copy(zerobuf, accum.at[pl.ds(base + b * CHUNK, CHUNK)])
    plsc.subcore_barrier()

    edge_base = sid * EDGES_PER_TILE
    iota16 = lax.iota(jnp.int32, 16)

    def _start_gathers(i, b):
        off = edge_base + i * CHUNK
        pltpu.sync_copy(ei_hbm.at[:, pl.ds(off, CHUNK)], ebuf[b])
        for j in range(CHUNK // 16):
            sl = pl.ds(j * 16, 16)
            s_v = ebuf[b][0, sl]
            d_v = ebuf[b][1, sl]
            gsidx[b][sl] = s_v + node_off
            gdidx[b][sl] = d_v + node_off
            didx[b][sl] = d_v
        pltpu.async_copy(kv_hbm.at[gsidx[b]], kvbuf[b], sem_g[b])
        pltpu.async_copy(q_hbm.at[gdidx[b]], qbuf[b], sem_g[b])

    def _wait_gathers(b):
        pltpu.make_async_copy(kv_hbm.at[gsidx[b]], kvbuf[b], sem_g[b]).wait()
        pltpu.make_async_copy(q_hbm.at[gdidx[b]], qbuf[b], sem_g[b]).wait()

    lanesel = [(iota16 // 4) == hh for hh in range(1, HPC)]

    def _compute(b):
        def _edge(e2, _):
            for u in range(4):
                e = 4 * e2 + u
                raws = []
                for hh in range(HPC):
                    cb = hh * HEAD_DIM
                    kvv = kvbuf[b][e, pl.ds(cb, 16)]
                    qvv = qbuf[b][e, pl.ds(cb, 16)]
                    raws.append(jnp.sum(kvv * qvv))
                comb = jnp.full((16,), raws[0], jnp.float32)
                for hh in range(1, HPC):
                    comb = jnp.where(lanesel[hh - 1],
                                     jnp.full((16,), raws[hh], jnp.float32), comb)
                s_all = jnp.exp(jnp.clip(comb, -5.0, 5.0))
                wvbuf[b][e, pl.ds(CW, 16)] = s_all
                for hh in range(HPC):
                    cb = hh * HEAD_DIM
                    sh = jnp.full((16,), s_all[4 * hh], jnp.float32)
                    vv = kvbuf[b][e, pl.ds(CW + cb, 16)]
                    wvbuf[b][e, pl.ds(cb, 16)] = vv * sh
            return 0

        lax.fori_loop(0, CHUNK // 4, _edge, 0)

    def _wait_scatter(b):
        pltpu.make_async_copy(wvbuf[b], accum.at[didx[b]], sem_s[b]).wait()

    def _issue_scatter(b):
        pltpu.async_copy(wvbuf[b], accum.at[didx[b]], sem_s[b], add=True)

    _start_gathers(0, 0)
    _start_gathers(1, 1)

    NSUP = N_CHUNKS // NBUF

    def _super(t, _):
        i0 = t * NBUF
        for b in range(NBUF):
            _wait_gathers(b)
            _compute(b)
            _issue_scatter(b)
            bj = (b + 2) % NBUF
            if b == 0:
                @pl.when(t > 0)
                def _():
                    _wait_scatter(bj)
                _start_gathers(i0 + 2, bj)
            elif b == NBUF - 1:
                @pl.when(t < NSUP - 1)
                def _():
                    _wait_scatter(bj)
                    _start_gathers(i0 + b + 2, bj)
            else:
                _wait_scatter(bj)
                _start_gathers(i0 + b + 2, bj)
        return 0

    lax.fori_loop(0, NSUP, _super, 0)
    _wait_gathers(0)
    _compute(0)
    _issue_scatter(0)
    for b in range(NBUF):
        _wait_scatter(b)
    plsc.subcore_barrier()

    for b in range(5):
        rs = pl.ds(base + b * 128, 128)
        pltpu.sync_copy(accum.at[rs], out_hbm.at[cid, rs])


def _edges(q2, kv2, ei32):
    mesh = plsc.VectorSubcoreMesh(core_axis_name="c", subcore_axis_name="s")
    idx_t = pltpu.VMEM((CHUNK,), jnp.int32)
    f = functools.partial(
        pl.kernel,
        out_type=jax.ShapeDtypeStruct((N_CORES, ACC_ROWS, ROW), jnp.float32),
        mesh=mesh,
        compiler_params=pltpu.CompilerParams(
            needs_layout_passes=False, use_tc_tiling_on_sc=False),
        scratch_types=[
            [pltpu.VMEM((2, CHUNK), jnp.int32)] * NBUF,
            [idx_t] * NBUF,
            [idx_t] * NBUF,
            [idx_t] * NBUF,
            [pltpu.VMEM((CHUNK, KVW), jnp.float32)] * NBUF,
            [pltpu.VMEM((CHUNK, CW), jnp.float32)] * NBUF,
            [pltpu.VMEM((CHUNK, ROW), jnp.float32)] * NBUF,
            pltpu.VMEM((CHUNK, ROW), jnp.float32),
            pltpu.VMEM_SHARED((ACC_ROWS, ROW), jnp.float32),
            [pltpu.SemaphoreType.DMA] * NBUF,
            [pltpu.SemaphoreType.DMA] * NBUF,
        ],
    )(_edge_body)
    return f(q2, kv2, ei32)


def _comb_body(p0_ref, p1_ref, b_ref, o_ref):
    bm = b_ref[...]
    s0 = p0_ref[...][0]
    s1 = p1_ref[...][0]
    z0 = jnp.dot(s0[:, CW:CW + 16], bm, preferred_element_type=jnp.float32)
    z1 = jnp.dot(s1[:, CW:CW + 16], bm, preferred_element_type=jnp.float32)
    o_ref[...] = jnp.concatenate([s0[:, :CW] / z0, s1[:, :CW] / z1], axis=1)


def _combine(partials):
    blk = 1000
    bm_np = np.zeros((16, CW), np.float32)
    for hh in range(HPC):
        bm_np[4 * hh, HEAD_DIM * hh:HEAD_DIM * (hh + 1)] = 1.0
    bmat = jnp.asarray(bm_np)
    return pl.pallas_call(
        _comb_body,
        grid=(N_NODES // blk,),
        in_specs=[
            pl.BlockSpec((1, blk, ROW), lambda i: (0, i, 0)),
            pl.BlockSpec((1, blk, ROW), lambda i: (1, i, 0)),
            pl.BlockSpec((16, CW), lambda i: (0, 0)),
        ],
        out_specs=pl.BlockSpec((blk, HD), lambda i: (i, 0)),
        out_shape=jax.ShapeDtypeStruct((N_NODES, HD), jnp.float32),
    )(partials, partials, bmat)


def kernel(h, edge_index, WQ, WK, WV):
    ei32 = edge_index.astype(jnp.int32)
    wq = (WQ * 0.25).reshape(IN_DIM, N_CORES, CW).transpose(1, 0, 2)
    wk = WK.reshape(IN_DIM, N_CORES, CW).transpose(1, 0, 2)
    wv = WV.reshape(IN_DIM, N_CORES, CW).transpose(1, 0, 2)
    q2, kv2 = _qkv(h, wq, wk, wv)
    partials = _edges(q2, kv2, ei32)
    out = _combine(partials)
    return out.reshape(N_NODES, NUM_HEADS, HEAD_DIM)
